# baseline (speedup 1.0000x reference)
import jax
import jax.numpy as jnp
import numpy as np
from jax.experimental import pallas as pl

STAGES = [128, 256]
HEADS = [8, 16]
KNN_K = [32, 32]
PF = [0.25, 0.25]
PH = [64, 64]
FF = [512, 1024]
OUT = [256, 64]


def _emb_body(x_ref, w_ref, b_ref, o_ref):
    o_ref[...] = jnp.dot(x_ref[...], w_ref[...],
                         preferred_element_type=jnp.float32) + b_ref[...]


def _emb(x, W, b):
    B, N, D = x.shape
    O = W.shape[1]
    return pl.pallas_call(
        _emb_body,
        out_shape=jax.ShapeDtypeStruct((B, N, O), jnp.float32),
        grid=(B,),
        in_specs=[
            pl.BlockSpec((1, N, D), lambda i: (i, 0, 0)),
            pl.BlockSpec((D, O), lambda i: (0, 0)),
            pl.BlockSpec((O,), lambda i: (0,)),
        ],
        out_specs=pl.BlockSpec((1, N, O), lambda i: (i, 0, 0)),
    )(x, W, b)


def _knn_idx(xv, K):
    n2 = jnp.sum(xv * xv, axis=-1)
    d2 = n2[:, :, None] + n2[:, None, :] - 2.0 * jnp.einsum('bnd,bmd->bnm', xv, xv)
    _, idx = jax.lax.top_k(-d2, K)
    return idx


def _gather(a, idx):
    return jax.vmap(lambda ab, ib: ab[ib])(a, idx)


def _mha_knn(x, xv, p, i, H, K):
    B, N, D = x.shape
    dh = D // H
    idx = _knn_idx(xv, K)
    q = x @ p[f'mha{i}_Wq'] + p[f'mha{i}_Wqb']
    k = x @ p[f'mha{i}_Wk'] + p[f'mha{i}_Wkb']
    v = x @ p[f'mha{i}_Wv'] + p[f'mha{i}_Wvb']
    kn = _gather(k, idx).reshape(B, N, K, H, dh)
    vn = _gather(v, idx).reshape(B, N, K, H, dh)
    qh = q.reshape(B, N, H, dh)
    logits = jnp.einsum('bnhd,bnkhd->bnhk', qh, kn) / np.sqrt(dh)
    w = jax.nn.softmax(logits, axis=-1)
    o = jnp.einsum('bnhk,bnkhd->bnhd', w, vn).reshape(B, N, D)
    return x + (o @ p[f'mha{i}_Wo'] + p[f'mha{i}_Wob'])


def kernel(x, x_v, params):
    x = _emb(x, params['emb_W'], params['emb_b'])
    extras = []
    for i in range(len(STAGES)):
        x = _mha_knn(x, x_v, params, i, HEADS[i], KNN_K[i])
        x = x + (jax.nn.relu(x @ params[f'ff{i}_W1'] + params[f'ff{i}_b1']) @ params[f'ff{i}_W2'] + params[f'ff{i}_b2'])
        h = jnp.tanh(x @ params[f'pool{i}_W1'] + params[f'pool{i}_b1'])
        s = (h @ params[f'pool{i}_W2'] + params[f'pool{i}_b2'])[..., 0]
        M = int(x.shape[1] * PF[i])
        vals, idx = jax.lax.top_k(s, M)
        x = _gather(x, idx) * jax.nn.sigmoid(vals)[..., None]
        x_v = _gather(x_v, idx)
        extras.append((vals, idx))
        x = x @ params[f'out{i}_W'] + params[f'out{i}_b']
    return (x, x_v, extras[0][0], extras[0][1], extras[1][0], extras[1][1])


# speedup vs baseline: 7.1945x; 7.1945x over previous
import functools

import jax
import jax.numpy as jnp
import numpy as np
from jax.experimental import pallas as pl

_STAGES = [128, 256]
_HEADS = [8, 16]
_KNN = 32
_PF = [0.25, 0.25]
_FF = [512, 1024]
_OUT = [256, 64]

_NEG_INF = np.float32(-np.inf)
_HI = jax.lax.Precision.HIGHEST


def _eye128():
    return (jax.lax.broadcasted_iota(jnp.int32, (128, 128), 0)
            == jax.lax.broadcasted_iota(jnp.int32, (128, 128), 1)
            ).astype(jnp.float32)


def _lt128():
    return (jax.lax.broadcasted_iota(jnp.int32, (128, 128), 0)
            < jax.lax.broadcasted_iota(jnp.int32, (128, 128), 1)
            ).astype(jnp.float32)


def _rowvec(col, eye):
    N = col.shape[0]
    dn = (((0,), (0,)), ((), ()))
    chunks = [jax.lax.dot_general(col[i * 128:(i + 1) * 128], eye, dn,
                                  precision=_HI,
                                  preferred_element_type=jnp.float32)
              for i in range(N // 128)]
    return jnp.concatenate(chunks, axis=1)


def _prefix_count(eq, lt):
    QB, N = eq.shape
    dn = (((1,), (0,)), ((), ()))
    carry = jnp.zeros((QB, 1), jnp.float32)
    outs = []
    for c in range(N // 128):
        blk = eq[:, c * 128:(c + 1) * 128]
        pref = carry + jax.lax.dot_general(blk, lt, dn,
                                           preferred_element_type=jnp.float32)
        outs.append(pref)
        carry = carry + jnp.sum(blk, axis=1, keepdims=True)
    return jnp.concatenate(outs, axis=1)


def _emb_body(x_ref, w_ref, b_ref, o_ref):
    o_ref[0] = jnp.dot(x_ref[0], w_ref[...],
                       preferred_element_type=jnp.float32) + b_ref[...]


def _emb(x, W, b):
    B, N, D = x.shape
    O = W.shape[1]
    return pl.pallas_call(
        _emb_body,
        out_shape=jax.ShapeDtypeStruct((B, N, O), jnp.float32),
        grid=(B,),
        in_specs=[
            pl.BlockSpec((1, N, D), lambda i: (i, 0, 0)),
            pl.BlockSpec((D, O), lambda i: (0, 0)),
            pl.BlockSpec((O,), lambda i: (0,)),
        ],
        out_specs=pl.BlockSpec((1, N, O), lambda i: (i, 0, 0)),
    )(x, W, b)


def _qkv_body(x_ref, wq, bq, wk, bk, wv, bv, q_ref, k_ref, v_ref):
    x = x_ref[0]
    q_ref[0] = jnp.dot(x, wq[...], preferred_element_type=jnp.float32) + bq[...]
    k_ref[0] = jnp.dot(x, wk[...], preferred_element_type=jnp.float32) + bk[...]
    v_ref[0] = jnp.dot(x, wv[...], preferred_element_type=jnp.float32) + bv[...]


def _qkv(x, wq, bq, wk, bk, wv, bv):
    B, N, D = x.shape
    wspec = pl.BlockSpec((D, D), lambda i: (0, 0))
    bspec = pl.BlockSpec((D,), lambda i: (0,))
    xspec = pl.BlockSpec((1, N, D), lambda i: (i, 0, 0))
    return pl.pallas_call(
        _qkv_body,
        out_shape=[jax.ShapeDtypeStruct((B, N, D), jnp.float32)] * 3,
        grid=(B,),
        in_specs=[xspec, wspec, bspec, wspec, bspec, wspec, bspec],
        out_specs=[xspec, xspec, xspec],
    )(x, wq, bq, wk, bk, wv, bv)


def _attn_body(H, K, xvq_ref, xva_ref, q_ref, k_ref, v_ref, x_ref,
               wo_ref, bo_ref, o_ref):
    QB = q_ref.shape[1]
    N = k_ref.shape[1]
    D = q_ref.shape[2]
    dh = D // H

    xvq = xvq_ref[0]
    xva = xva_ref[0]
    n2q = jnp.sum(xvq * xvq, axis=1, keepdims=True)
    n2a_col = jnp.sum(xva * xva, axis=1, keepdims=True)
    n2a_row = _rowvec(n2a_col, _eye128())
    e = jax.lax.dot_general(xvq, xva, (((1,), (1,)), ((), ())),
                            preferred_element_type=jnp.float32)
    d2 = (n2q + n2a_row) - 2.0 * e

    bits = jax.lax.bitcast_convert_type(d2, jnp.int32)
    key = jnp.where(bits >= 0, bits, -(bits & jnp.int32(0x7FFFFFFF)))

    def bs_step(_, lohi):
        lo, hi = lohi
        mid = (lo & hi) + ((lo ^ hi) >> 1)
        cnt = jnp.sum((key <= mid).astype(jnp.int32), axis=1, keepdims=True)
        ge = cnt >= K
        return jnp.where(ge, lo, mid + 1), jnp.where(ge, mid, hi)

    lo0 = jnp.full((QB, 1), jnp.iinfo(jnp.int32).min, jnp.int32)
    hi0 = jnp.full((QB, 1), jnp.iinfo(jnp.int32).max, jnp.int32)
    lo, _ = jax.lax.fori_loop(0, 32, bs_step, (lo0, hi0))
    less = key < lo
    eq = key == lo
    need = (K - jnp.sum(less.astype(jnp.int32), axis=1, keepdims=True)
            ).astype(jnp.float32)
    eq_pref = _prefix_count(eq.astype(jnp.float32), _lt128())
    mask = less | (eq & (eq_pref < need))

    q = q_ref[0]
    k = k_ref[0]
    lane = jax.lax.broadcasted_iota(jnp.int32, (1, D), 1)
    scale = jnp.float32(1.0 / np.sqrt(dh))
    ohs = []
    for h in range(H):
        colmask = (lane >= h * dh) & (lane < (h + 1) * dh)
        qm = jnp.where(colmask, q, 0.0)
        L = jax.lax.dot_general(qm, k, (((1,), (1,)), ((), ())),
                                precision=_HI,
                                preferred_element_type=jnp.float32) * scale
        Lm = jnp.where(mask, L, _NEG_INF)
        mx = jnp.max(Lm, axis=1, keepdims=True)
        p = jnp.exp(Lm - mx)
        den = jnp.sum(p, axis=1, keepdims=True)
        w = p / den
        vh = v_ref[0, :, h * dh:(h + 1) * dh]
        ohs.append(jax.lax.dot_general(w, vh, (((1,), (0,)), ((), ())),
                                       precision=_HI,
                                       preferred_element_type=jnp.float32))
    o = jnp.concatenate(ohs, axis=1)
    o_ref[0] = x_ref[0] + (jnp.dot(o, wo_ref[...],
                                   preferred_element_type=jnp.float32)
                           + bo_ref[...])


def _attn(x, xv, q, k, v, wo, bo, H, QB):
    B, N, D = x.shape
    kernel = functools.partial(_attn_body, H, _KNN)
    return pl.pallas_call(
        kernel,
        out_shape=jax.ShapeDtypeStruct((B, N, D), jnp.float32),
        grid=(B, N // QB),
        in_specs=[
            pl.BlockSpec((1, QB, 3), lambda i, j: (i, j, 0)),
            pl.BlockSpec((1, N, 3), lambda i, j: (i, 0, 0)),
            pl.BlockSpec((1, QB, D), lambda i, j: (i, j, 0)),
            pl.BlockSpec((1, N, D), lambda i, j: (i, 0, 0)),
            pl.BlockSpec((1, N, D), lambda i, j: (i, 0, 0)),
            pl.BlockSpec((1, QB, D), lambda i, j: (i, j, 0)),
            pl.BlockSpec((D, D), lambda i, j: (0, 0)),
            pl.BlockSpec((D,), lambda i, j: (0,)),
        ],
        out_specs=pl.BlockSpec((1, QB, D), lambda i, j: (i, j, 0)),
    )(xv, xv, q, k, v, x, wo, bo)


def _ffpool_body(x_ref, w1, b1, w2, b2, p1, pb1, p2, pb2, xo_ref, s_ref):
    x = x_ref[0]
    t = jnp.maximum(jnp.dot(x, w1[...], preferred_element_type=jnp.float32)
                    + b1[...], 0.0)
    x2 = x + (jnp.dot(t, w2[...], preferred_element_type=jnp.float32)
              + b2[...])
    h = jnp.tanh(jnp.dot(x2, p1[...], preferred_element_type=jnp.float32)
                 + pb1[...])
    s = jnp.dot(h, p2[...], preferred_element_type=jnp.float32) + pb2[...]
    xo_ref[0] = x2
    s_ref[0] = s


def _ffpool(x, w1, b1, w2, b2, p1, pb1, p2, pb2):
    B, N, D = x.shape
    F = w1.shape[1]
    PH = p1.shape[1]
    return pl.pallas_call(
        _ffpool_body,
        out_shape=[jax.ShapeDtypeStruct((B, N, D), jnp.float32),
                   jax.ShapeDtypeStruct((B, N, 1), jnp.float32)],
        grid=(B,),
        in_specs=[
            pl.BlockSpec((1, N, D), lambda i: (i, 0, 0)),
            pl.BlockSpec((D, F), lambda i: (0, 0)),
            pl.BlockSpec((F,), lambda i: (0,)),
            pl.BlockSpec((F, D), lambda i: (0, 0)),
            pl.BlockSpec((D,), lambda i: (0,)),
            pl.BlockSpec((D, PH), lambda i: (0, 0)),
            pl.BlockSpec((PH,), lambda i: (0,)),
            pl.BlockSpec((PH, 1), lambda i: (0, 0)),
            pl.BlockSpec((1,), lambda i: (0,)),
        ],
        out_specs=[pl.BlockSpec((1, N, D), lambda i: (i, 0, 0)),
                   pl.BlockSpec((1, N, 1), lambda i: (i, 0, 0))],
    )(x, w1, b1, w2, b2, p1, pb1, p2, pb2)


def _topk_body(M, CB, s_ref, x_ref, xv_ref, w_ref, b_ref,
               vals_ref, idx_ref, xn_ref, xvn_ref):
    N = s_ref.shape[1]
    D = x_ref.shape[2]
    s_col = s_ref[0]
    s_row = _rowvec(s_col, _eye128())
    lane_n = jax.lax.broadcasted_iota(jnp.int32, (1, N), 1)
    lane_m = jax.lax.broadcasted_iota(jnp.int32, (1, M), 1)

    vals = jnp.zeros((M, 1), jnp.float32)
    idxf = jnp.zeros((M, 1), jnp.float32)
    xg = jnp.zeros((M, D), jnp.float32)
    xvg = jnp.zeros((M, 3), jnp.float32)
    for c in range(N // CB):
        s_j = s_col[c * CB:(c + 1) * CB]
        jg = (jax.lax.broadcasted_iota(jnp.int32, (CB, 1), 0)
              + jnp.int32(c * CB))
        gt = jnp.sum((s_row > s_j).astype(jnp.int32), axis=1, keepdims=True)
        eq = jnp.sum(((s_row == s_j) & (lane_n < jg)).astype(jnp.int32),
                     axis=1, keepdims=True)
        rank = gt + eq
        oh = ((rank == lane_m) & (rank < M)).astype(jnp.float32)
        dn = (((0,), (0,)), ((), ()))
        vals = vals + jax.lax.dot_general(oh, s_j, dn, precision=_HI,
                                          preferred_element_type=jnp.float32)
        idxf = idxf + jax.lax.dot_general(oh, jg.astype(jnp.float32), dn,
                                          precision=_HI,
                                          preferred_element_type=jnp.float32)
        xg = xg + jax.lax.dot_general(oh, x_ref[0, c * CB:(c + 1) * CB, :], dn,
                                      precision=_HI,
                                      preferred_element_type=jnp.float32)
        xvg = xvg + jax.lax.dot_general(oh, xv_ref[0, c * CB:(c + 1) * CB, :],
                                        dn, precision=_HI,
                                        preferred_element_type=jnp.float32)
    sig = jax.nn.sigmoid(vals)
    xs = xg * sig
    vals_ref[0] = vals
    idx_ref[0] = idxf.astype(jnp.int32)
    xn_ref[0] = jnp.dot(xs, w_ref[...], preferred_element_type=jnp.float32) \
        + b_ref[...]
    xvn_ref[0] = xvg


def _topk_pool(s, x, xv, outW, outb, M, CB):
    B, N, D = x.shape
    O = outW.shape[1]
    kernel = functools.partial(_topk_body, M, CB)
    return pl.pallas_call(
        kernel,
        out_shape=[jax.ShapeDtypeStruct((B, M, 1), jnp.float32),
                   jax.ShapeDtypeStruct((B, M, 1), jnp.int32),
                   jax.ShapeDtypeStruct((B, M, O), jnp.float32),
                   jax.ShapeDtypeStruct((B, M, 3), jnp.float32)],
        grid=(B,),
        in_specs=[
            pl.BlockSpec((1, N, 1), lambda i: (i, 0, 0)),
            pl.BlockSpec((1, N, D), lambda i: (i, 0, 0)),
            pl.BlockSpec((1, N, 3), lambda i: (i, 0, 0)),
            pl.BlockSpec((D, O), lambda i: (0, 0)),
            pl.BlockSpec((O,), lambda i: (0,)),
        ],
        out_specs=[pl.BlockSpec((1, M, 1), lambda i: (i, 0, 0)),
                   pl.BlockSpec((1, M, 1), lambda i: (i, 0, 0)),
                   pl.BlockSpec((1, M, O), lambda i: (i, 0, 0)),
                   pl.BlockSpec((1, M, 3), lambda i: (i, 0, 0))],
    )(s, x, xv, outW, outb)


def kernel(x, x_v, params):
    p = params
    x = _emb(x, p['emb_W'], p['emb_b'])
    xv = x_v
    extras = []
    for i in range(len(_STAGES)):
        B, N, D = x.shape
        q, k, v = _qkv(x, p[f'mha{i}_Wq'], p[f'mha{i}_Wqb'],
                       p[f'mha{i}_Wk'], p[f'mha{i}_Wkb'],
                       p[f'mha{i}_Wv'], p[f'mha{i}_Wvb'])
        x = _attn(x, xv, q, k, v, p[f'mha{i}_Wo'], p[f'mha{i}_Wob'],
                  _HEADS[i], 256)
        x, s = _ffpool(x, p[f'ff{i}_W1'], p[f'ff{i}_b1'],
                       p[f'ff{i}_W2'], p[f'ff{i}_b2'],
                       p[f'pool{i}_W1'], p[f'pool{i}_b1'],
                       p[f'pool{i}_W2'], p[f'pool{i}_b2'])
        M = int(N * _PF[i])
        vals, idx, x, xv = _topk_pool(s, x, xv, p[f'out{i}_W'], p[f'out{i}_b'],
                                      M, 256)
        extras.append((vals[..., 0], idx[..., 0]))
    return (x, xv, extras[0][0], extras[0][1], extras[1][0], extras[1][1])
